# baseline (speedup 1.0000x reference)
import jax
import jax.numpy as jnp
from jax.experimental import pallas as pl
from jax.experimental.pallas import tpu as pltpu

_OUT_SIZE = 9
_SUB_S = 16
_SUB_Q = 256


def _produce(nu_ref, b1_ref, ncsel_scr, bsel_scr, nc_scr, g0_scr, g1_scr,
             c_scr):
    lam_dim, s_blk, nf = nu_ref.shape
    l_dim = b1_ref.shape[0]
    n_ch = l_dim * _OUT_SIZE

    nus = [nu_ref[x] for x in range(lam_dim)]
    for y in range(l_dim):
        for mm in range(_OUT_SIZE):
            acc = None
            for x in range(lam_dim):
                t = nus[x] * c_scr[(x * l_dim + y) * _OUT_SIZE + mm]
                acc = t if acc is None else acc + t
            ch = y * _OUT_SIZE + mm
            nc_scr[ch * s_blk:(ch + 1) * s_blk, :] = acc.astype(jnp.bfloat16)

    for y in range(l_dim):
        base = y * _OUT_SIZE * s_blk
        ncsel_scr[base:base + _OUT_SIZE * s_blk, :] = jnp.dot(
            nc_scr[base:base + _OUT_SIZE * s_blk, :], g0_scr[...],
            preferred_element_type=jnp.float32).astype(jnp.bfloat16)
    bsel_scr[...] = jnp.dot(
        b1_ref[...].astype(jnp.bfloat16).reshape(l_dim * s_blk, nf),
        g1_scr[...], preferred_element_type=jnp.float32).astype(jnp.bfloat16)


def _combine(ncsel_scr, bsel_scr, out_ref):
    _, s_blk, q = out_ref.shape
    l_dim = bsel_scr.shape[0] // s_blk

    def sub_body(j, carry):
        row = j * _SUB_S
        for qh in range(q // _SUB_Q):
            qs = qh * _SUB_Q
            bys = [bsel_scr[pl.ds(y * s_blk + row, _SUB_S),
                            qs:qs + _SUB_Q].astype(jnp.float32)
                   for y in range(l_dim)]
            for mm in range(_OUT_SIZE):
                terms = []
                for y in range(l_dim):
                    ch = y * _OUT_SIZE + mm
                    nct = ncsel_scr[pl.ds(ch * s_blk + row, _SUB_S),
                                    qs:qs + _SUB_Q].astype(jnp.float32)
                    terms.append(nct * bys[y])
                while len(terms) > 1:
                    terms = [a + b for a, b in zip(terms[::2], terms[1::2])] \
                        + ([terms[-1]] if len(terms) % 2 else [])
                out_ref[mm, pl.ds(row, _SUB_S), qs:qs + _SUB_Q] = terms[0]
        return carry

    jax.lax.fori_loop(0, s_blk // _SUB_S, sub_body, 0)


def _combine_body(mu_ref, m_ref, M_ref, cg_ref, nu_ref, b1_ref, sel0_ref,
                  sel1_ref, out_ref, nc_scr, ncsel_a, ncsel_b, bsel_a,
                  bsel_b, g0_scr, g1_scr, c_scr):
    lam_dim, s_blk, nf = nu_ref.shape
    l_dim = b1_ref.shape[0]
    q = sel0_ref.shape[1]
    i = pl.program_id(0)
    n_terms = mu_ref.shape[0]

    @pl.when(i == 0)
    def _():
        def zero_body(k, carry):
            c_scr[k] = 0.0
            return carry
        jax.lax.fori_loop(0, lam_dim * l_dim * _OUT_SIZE, zero_body, 0)

        def scat_body(t, carry):
            idx = mu_ref[t] * (l_dim * _OUT_SIZE) + m_ref[t] * _OUT_SIZE + M_ref[t]
            c_scr[idx] = c_scr[idx] + cg_ref[t]
            return carry
        jax.lax.fori_loop(0, n_terms, scat_body, 0)

        iota_f = jax.lax.broadcasted_iota(jnp.int32, (nf, q), 0)
        g0_scr[...] = (iota_f == sel0_ref[...]).astype(jnp.bfloat16)
        g1_scr[...] = (iota_f == sel1_ref[...]).astype(jnp.bfloat16)

    even = jax.lax.rem(i, 2) == 0

    @pl.when(even)
    def _():
        _produce(nu_ref, b1_ref, ncsel_a, bsel_a, nc_scr, g0_scr, g1_scr,
                 c_scr)
        _combine(ncsel_b, bsel_b, out_ref)

    @pl.when(jnp.logical_not(even))
    def _():
        _produce(nu_ref, b1_ref, ncsel_b, bsel_b, nc_scr, g0_scr, g1_scr,
                 c_scr)
        _combine(ncsel_a, bsel_a, out_ref)


def kernel(block_nu_values, block_1_values, selected_features, mu_array,
           m_array, M_array, cg_array):
    n, lam_dim, _ = block_nu_values.shape
    l_dim = block_1_values.shape[1]
    nf = block_1_values.shape[2]
    q = selected_features.shape[0]

    s_blk = 80
    assert n % s_blk == 0
    n_blk = n // s_blk
    grid = (n_blk + 1,)

    nu_t = block_nu_values[:, :, :nf].transpose(1, 0, 2)
    b1_t = block_1_values.transpose(1, 0, 2)
    sel0 = selected_features[:, 0].reshape(1, q).astype(jnp.int32)
    sel1 = selected_features[:, 1].reshape(1, q).astype(jnp.int32)

    last = n_blk - 1
    grid_spec = pltpu.PrefetchScalarGridSpec(
        num_scalar_prefetch=4,
        grid=grid,
        in_specs=[
            pl.BlockSpec((lam_dim, s_blk, nf),
                         lambda i, *_: (0, jnp.minimum(i, last), 0)),
            pl.BlockSpec((l_dim, s_blk, nf),
                         lambda i, *_: (0, jnp.minimum(i, last), 0)),
            pl.BlockSpec((1, q), lambda i, *_: (0, 0)),
            pl.BlockSpec((1, q), lambda i, *_: (0, 0)),
        ],
        out_specs=pl.BlockSpec((_OUT_SIZE, s_blk, q),
                               lambda i, *_: (0, jnp.maximum(i - 1, 0), 0)),
        scratch_shapes=[
            pltpu.VMEM((l_dim * _OUT_SIZE * s_blk, nf), jnp.bfloat16),
            pltpu.VMEM((l_dim * _OUT_SIZE * s_blk, q), jnp.bfloat16),
            pltpu.VMEM((l_dim * _OUT_SIZE * s_blk, q), jnp.bfloat16),
            pltpu.VMEM((l_dim * s_blk, q), jnp.bfloat16),
            pltpu.VMEM((l_dim * s_blk, q), jnp.bfloat16),
            pltpu.VMEM((nf, q), jnp.bfloat16),
            pltpu.VMEM((nf, q), jnp.bfloat16),
            pltpu.SMEM((lam_dim * l_dim * _OUT_SIZE,), jnp.float32),
        ],
    )
    out_t = pl.pallas_call(
        _combine_body,
        grid_spec=grid_spec,
        out_shape=jax.ShapeDtypeStruct((_OUT_SIZE, n, q), jnp.float32),
        compiler_params=pltpu.CompilerParams(
            dimension_semantics=("arbitrary",),
        ),
    )(mu_array, m_array, M_array, cg_array, nu_t, b1_t, sel0, sel1)
    return out_t.transpose(1, 0, 2)


# speedup vs baseline: 1.1023x; 1.1023x over previous
import jax
import jax.numpy as jnp
from jax.experimental import pallas as pl
from jax.experimental.pallas import tpu as pltpu

_OUT_SIZE = 9
_SUB_S = 16
_SUB_Q = 256


def _combine_body(mu_ref, m_ref, M_ref, cg_ref, nu_ref, b1_ref, sel0_ref,
                  sel1_ref, out_ref, nc_scr, ncsel_scr, bsel_scr, g0_scr,
                  g1_scr, c_scr):
    lam_dim, s_blk, nf = nu_ref.shape
    l_dim = b1_ref.shape[0]
    q = sel0_ref.shape[1]
    i = pl.program_id(0)
    n_terms = mu_ref.shape[0]

    @pl.when(i == 0)
    def _():
        def zero_body(k, carry):
            c_scr[k] = 0.0
            return carry
        jax.lax.fori_loop(0, lam_dim * l_dim * _OUT_SIZE, zero_body, 0)

        def scat_body(t, carry):
            idx = mu_ref[t] * (l_dim * _OUT_SIZE) + m_ref[t] * _OUT_SIZE + M_ref[t]
            c_scr[idx] = c_scr[idx] + cg_ref[t]
            return carry
        jax.lax.fori_loop(0, n_terms, scat_body, 0)

        iota_f = jax.lax.broadcasted_iota(jnp.int32, (nf, q), 0)
        g0_scr[...] = (iota_f == sel0_ref[...]).astype(jnp.bfloat16)
        g1_scr[...] = (iota_f == sel1_ref[...]).astype(jnp.bfloat16)

    nus = [nu_ref[x] for x in range(lam_dim)]
    for y in range(l_dim):
        for mm in range(_OUT_SIZE):
            acc = None
            for x in range(lam_dim):
                t = nus[x] * c_scr[(x * l_dim + y) * _OUT_SIZE + mm]
                acc = t if acc is None else acc + t
            ch = y * _OUT_SIZE + mm
            nc_scr[ch * s_blk:(ch + 1) * s_blk, :] = acc.astype(jnp.bfloat16)

    for y in range(l_dim):
        base = y * _OUT_SIZE * s_blk
        ncsel_scr[base:base + _OUT_SIZE * s_blk, :] = jnp.dot(
            nc_scr[base:base + _OUT_SIZE * s_blk, :], g0_scr[...],
            preferred_element_type=jnp.float32).astype(jnp.bfloat16)
    bsel_scr[...] = jnp.dot(
        b1_ref[...].astype(jnp.bfloat16).reshape(l_dim * s_blk, nf),
        g1_scr[...], preferred_element_type=jnp.float32).astype(jnp.bfloat16)

    def sub_body(j, carry):
        row = j * _SUB_S
        for qh in range(q // _SUB_Q):
            qs = qh * _SUB_Q
            bys = [bsel_scr[pl.ds(y * s_blk + row, _SUB_S),
                            qs:qs + _SUB_Q].astype(jnp.float32)
                   for y in range(l_dim)]
            for mm in range(_OUT_SIZE):
                acc = None
                for y in range(l_dim):
                    ch = y * _OUT_SIZE + mm
                    nct = ncsel_scr[pl.ds(ch * s_blk + row, _SUB_S),
                                    qs:qs + _SUB_Q].astype(jnp.float32)
                    t = nct * bys[y]
                    acc = t if acc is None else acc + t
                out_ref[mm, pl.ds(row, _SUB_S), qs:qs + _SUB_Q] = acc
        return carry

    jax.lax.fori_loop(0, s_blk // _SUB_S, sub_body, 0)


def kernel(block_nu_values, block_1_values, selected_features, mu_array,
           m_array, M_array, cg_array):
    n, lam_dim, _ = block_nu_values.shape
    l_dim = block_1_values.shape[1]
    nf = block_1_values.shape[2]
    q = selected_features.shape[0]

    s_blk = 80
    assert n % s_blk == 0
    grid = (n // s_blk,)

    nu_t = block_nu_values[:, :, :nf].transpose(1, 0, 2)
    b1_t = block_1_values.transpose(1, 0, 2)
    sel0 = selected_features[:, 0].reshape(1, q).astype(jnp.int32)
    sel1 = selected_features[:, 1].reshape(1, q).astype(jnp.int32)

    grid_spec = pltpu.PrefetchScalarGridSpec(
        num_scalar_prefetch=4,
        grid=grid,
        in_specs=[
            pl.BlockSpec((lam_dim, s_blk, nf), lambda i, *_: (0, i, 0)),
            pl.BlockSpec((l_dim, s_blk, nf), lambda i, *_: (0, i, 0)),
            pl.BlockSpec((1, q), lambda i, *_: (0, 0)),
            pl.BlockSpec((1, q), lambda i, *_: (0, 0)),
        ],
        out_specs=pl.BlockSpec((_OUT_SIZE, s_blk, q), lambda i, *_: (0, i, 0)),
        scratch_shapes=[
            pltpu.VMEM((l_dim * _OUT_SIZE * s_blk, nf), jnp.bfloat16),
            pltpu.VMEM((l_dim * _OUT_SIZE * s_blk, q), jnp.bfloat16),
            pltpu.VMEM((l_dim * s_blk, q), jnp.bfloat16),
            pltpu.VMEM((nf, q), jnp.bfloat16),
            pltpu.VMEM((nf, q), jnp.bfloat16),
            pltpu.SMEM((lam_dim * l_dim * _OUT_SIZE,), jnp.float32),
        ],
    )
    out_t = pl.pallas_call(
        _combine_body,
        grid_spec=grid_spec,
        out_shape=jax.ShapeDtypeStruct((_OUT_SIZE, n, q), jnp.float32),
        compiler_params=pltpu.CompilerParams(
            dimension_semantics=("arbitrary",),
        ),
    )(mu_array, m_array, M_array, cg_array, nu_t, b1_t, sel0, sel1)
    return out_t.transpose(1, 0, 2)
